# baseline (speedup 1.0000x reference)
import functools

import jax
import jax.numpy as jnp
import numpy as np
from jax import lax
from jax.experimental import pallas as pl
from jax.experimental.pallas import tpu as pltpu
from jax.experimental.pallas import tpu_sc as plsc

A = 128
E = 16
M = 32
N = 10000
NE = N * M
D = 2 * A
EPS = 1e-5
LOG2E = 1.4426950408889634

NCHUNKS = 5
NHALF = N // NCHUNKS
NEH = NHALF * M

NW = 32


def _softplus(x):
    return jnp.maximum(x, 0.0) + jnp.log(1.0 + jnp.exp(-jnp.abs(x)))


def _pack_halves(zf, zc):
    uf = jax.lax.bitcast_convert_type(zf, jnp.uint32)
    uc = jax.lax.bitcast_convert_type(zc, jnp.uint32)
    hi = (uf + np.uint32(0x8000)) & np.uint32(0xFFFF0000)
    lo = (uc + np.uint32(0x8000)) >> 16
    return hi | lo


def _unpack_halves(w):
    gf = jax.lax.bitcast_convert_type(w & np.uint32(0xFFFF0000), jnp.float32)
    gc = jax.lax.bitcast_convert_type(w << 16, jnp.float32)
    return gf, gc


def _prep_body(af_ref, w_ref, b_ref, a_ref, pn_ref):
    p = jnp.dot(af_ref[...], w_ref[...], preferred_element_type=jnp.float32)
    a_ref[...] = p[:, :D] + b_ref[...]
    pn = p[:, D:]
    pn_ref[...] = _pack_halves(pn[:, :A], pn[:, A:])


def _prep(atom_fea, w_cat, b2d):
    bn = 2000
    return pl.pallas_call(
        _prep_body,
        grid=(N // bn,),
        in_specs=[
            pl.BlockSpec((bn, A), lambda i: (i, 0)),
            pl.BlockSpec((A, 2 * D), lambda i: (0, 0)),
            pl.BlockSpec((1, D), lambda i: (0, 0)),
        ],
        out_specs=[
            pl.BlockSpec((bn, D), lambda i: (i, 0)),
            pl.BlockSpec((bn, A), lambda i: (i, 0)),
        ],
        out_shape=[
            jax.ShapeDtypeStruct((N, D), jnp.float32),
            jax.ShapeDtypeStruct((N, A), jnp.uint32),
        ],
    )(atom_fea, w_cat, b2d)


EPW = NEH // NW
KCH = 200
NCH = EPW // KCH


def _gather_body(idx_hbm, pn_hbm, out_hbm, idx_v, rows0, rows1, sem0, sem1):
    wid = lax.axis_index("s") * 2 + lax.axis_index("c")
    base = wid * EPW
    pltpu.sync_copy(idx_hbm.at[pl.ds(base, EPW)], idx_v)

    pltpu.async_copy(pn_hbm.at[idx_v.at[pl.ds(0, KCH)]], rows0, sem0)

    def pair(i, carry):
        c0 = 2 * i
        c1 = c0 + 1
        pltpu.async_copy(pn_hbm.at[idx_v.at[pl.ds(c1 * KCH, KCH)]], rows1, sem1)
        pltpu.make_async_copy(
            pn_hbm.at[idx_v.at[pl.ds(c0 * KCH, KCH)]], rows0, sem0
        ).wait()
        pltpu.sync_copy(rows0, out_hbm.at[pl.ds(base + c0 * KCH, KCH)])

        @pl.when(i + 1 < NCH // 2)
        def _():
            pltpu.async_copy(
                pn_hbm.at[idx_v.at[pl.ds((c0 + 2) * KCH, KCH)]], rows0, sem0
            )

        pltpu.make_async_copy(
            pn_hbm.at[idx_v.at[pl.ds(c1 * KCH, KCH)]], rows1, sem1
        ).wait()
        pltpu.sync_copy(rows1, out_hbm.at[pl.ds(base + c1 * KCH, KCH)])
        return carry

    lax.fori_loop(0, NCH // 2, pair, 0)


def _gather(idx_flat, pn):
    mesh = plsc.VectorSubcoreMesh(core_axis_name="c", subcore_axis_name="s")
    k = functools.partial(
        pl.kernel,
        out_type=jax.ShapeDtypeStruct((NEH, A), jnp.uint32),
        mesh=mesh,
        scratch_types=[
            pltpu.VMEM((EPW,), jnp.int32),
            pltpu.VMEM((KCH, A), jnp.uint32),
            pltpu.VMEM((KCH, A), jnp.uint32),
            pltpu.SemaphoreType.DMA,
            pltpu.SemaphoreType.DMA,
        ],
    )(_gather_body)
    return k(idx_flat, pn)


BN3 = 200
GRID3 = NHALF // BN3
BE3 = BN3 * M


def _make_stats(h):
    nb = h * (NHALF // BN3)

    def body(g_ref, nf_ref, a_ref, we_ref, s1_ref, s2_ref, acc1, acc2):
        gf, gc = _unpack_halves(g_ref[...])
        nf2 = jnp.reshape(nf_ref[...], (BE3, E))
        e = jnp.dot(nf2, we_ref[...], preferred_element_type=jnp.float32)
        a = a_ref[...]
        zf = jnp.reshape(
            gf + jnp.reshape(e[:, :A], (M, BN3, A)) + a[None, :, :A], (BE3, A))
        zc = jnp.reshape(
            gc + jnp.reshape(e[:, A:], (M, BN3, A)) + a[None, :, A:], (BE3, A))

        @pl.when(pl.program_id(0) == 0)
        def _():
            acc1[...] = jnp.zeros_like(acc1)
            acc2[...] = jnp.zeros_like(acc2)

        acc1[0:1, :] += jnp.sum(zf, axis=0, keepdims=True)
        acc1[1:2, :] += jnp.sum(zc, axis=0, keepdims=True)
        acc2[0:1, :] += jnp.sum(zf * zf, axis=0, keepdims=True)
        acc2[1:2, :] += jnp.sum(zc * zc, axis=0, keepdims=True)

        @pl.when(pl.program_id(0) == GRID3 - 1)
        def _():
            s1_ref[...] = acc1[...]
            s2_ref[...] = acc2[...]

    def call(g3, nft, a, we):
        return pl.pallas_call(
            body,
            grid=(GRID3,),
            in_specs=[
                pl.BlockSpec((M, BN3, A), lambda i: (0, i, 0)),
                pl.BlockSpec((M, BN3, E), lambda i: (0, i + nb, 0)),
                pl.BlockSpec((BN3, D), lambda i: (i + h * (NHALF // BN3), 0)),
                pl.BlockSpec((E, D), lambda i: (0, 0)),
            ],
            out_specs=[
                pl.BlockSpec((2, A), lambda i: (0, 0)),
                pl.BlockSpec((2, A), lambda i: (0, 0)),
            ],
            out_shape=[
                jax.ShapeDtypeStruct((2, A), jnp.float32),
                jax.ShapeDtypeStruct((2, A), jnp.float32),
            ],
            scratch_shapes=[
                pltpu.VMEM((2, A), jnp.float32),
                pltpu.VMEM((2, A), jnp.float32),
            ],
        )(g3, nft, a, we)

    return call


BN = 200
GRID4 = NHALF // BN
BE = BN * M


def _make_pass2(h):
    nb = h * (NHALF // BN)

    def body(g_ref, nf_ref, a_ref, we_ref, s1_ref, s2_ref, g1_ref, b1_ref,
             sm_ref, s3_ref, s4_ref, acc3, acc4):
        mu = s1_ref[...] * (1.0 / NE)
        var = s2_ref[...] * (1.0 / NE) - mu * mu
        s = g1_ref[...] * lax.rsqrt(var + EPS)
        t = b1_ref[...] - mu * s
        sx = jnp.concatenate([s[0:1, :] * (-LOG2E), s[1:2, :] * LOG2E], axis=1)
        tx = jnp.concatenate([t[0:1, :] * (-LOG2E), t[1:2, :] * LOG2E], axis=1)
        we_s = we_ref[...] * sx
        a2 = a_ref[...] * sx + tx

        gf, gc = _unpack_halves(g_ref[...])
        nf2 = jnp.reshape(nf_ref[...], (BE, E))
        e = jnp.dot(nf2, we_s, preferred_element_type=jnp.float32)
        yf3 = (gf * sx[:1, None, :A] + jnp.reshape(e[:, :A], (M, BN, A))
               + a2[None, :, :A])
        yc3 = (gc * sx[:1, None, A:] + jnp.reshape(e[:, A:], (M, BN, A))
               + a2[None, :, A:])
        prod = jnp.log2(1.0 + jnp.exp2(yc3)) / (1.0 + jnp.exp2(yf3))
        sm = jnp.sum(prod, axis=0)
        sm_ref[...] = sm

        @pl.when(pl.program_id(0) == 0)
        def _():
            acc3[...] = jnp.zeros_like(acc3)
            acc4[...] = jnp.zeros_like(acc4)

        acc3[...] += jnp.sum(sm, axis=0, keepdims=True)
        acc4[...] += jnp.sum(sm * sm, axis=0, keepdims=True)

        @pl.when(pl.program_id(0) == GRID4 - 1)
        def _():
            s3_ref[...] = acc3[...]
            s4_ref[...] = acc4[...]

    def call(g3, nft, a, we, s1, s2, g1, b1):
        return pl.pallas_call(
            body,
            grid=(GRID4,),
            in_specs=[
                pl.BlockSpec((M, BN, A), lambda i: (0, i, 0)),
                pl.BlockSpec((M, BN, E), lambda i: (0, i + nb, 0)),
                pl.BlockSpec((BN, D), lambda i: (i + h * (NHALF // BN), 0)),
                pl.BlockSpec((E, D), lambda i: (0, 0)),
                pl.BlockSpec((2, A), lambda i: (0, 0)),
                pl.BlockSpec((2, A), lambda i: (0, 0)),
                pl.BlockSpec((2, A), lambda i: (0, 0)),
                pl.BlockSpec((2, A), lambda i: (0, 0)),
            ],
            out_specs=[
                pl.BlockSpec((BN, A), lambda i: (i, 0)),
                pl.BlockSpec((1, A), lambda i: (0, 0)),
                pl.BlockSpec((1, A), lambda i: (0, 0)),
            ],
            out_shape=[
                jax.ShapeDtypeStruct((NHALF, A), jnp.float32),
                jax.ShapeDtypeStruct((1, A), jnp.float32),
                jax.ShapeDtypeStruct((1, A), jnp.float32),
            ],
            scratch_shapes=[
                pltpu.VMEM((1, A), jnp.float32),
                pltpu.VMEM((1, A), jnp.float32),
            ],
        )(g3, nft, a, we, s1, s2, g1, b1)

    return call


def _final_body(sm_ref, af_ref, s3_ref, s4_ref, g2_ref, b2_ref, out_ref):
    mu = s3_ref[...] * (1.0 / N)
    var = s4_ref[...] * (1.0 / N) - mu * mu
    s = g2_ref[...] * lax.rsqrt(var + EPS)
    t = b2_ref[...] - mu * s
    out_ref[...] = _softplus(af_ref[...] + sm_ref[...] * s + t)


def _final(sm, atom_fea, s3, s4, g2, b2):
    bn5 = 2000
    return pl.pallas_call(
        _final_body,
        grid=(N // bn5,),
        in_specs=[
            pl.BlockSpec((bn5, A), lambda i: (i, 0)),
            pl.BlockSpec((bn5, A), lambda i: (i, 0)),
            pl.BlockSpec((1, A), lambda i: (0, 0)),
            pl.BlockSpec((1, A), lambda i: (0, 0)),
            pl.BlockSpec((1, A), lambda i: (0, 0)),
            pl.BlockSpec((1, A), lambda i: (0, 0)),
        ],
        out_specs=pl.BlockSpec((bn5, A), lambda i: (i, 0)),
        out_shape=jax.ShapeDtypeStruct((N, A), jnp.float32),
    )(sm, atom_fea, s3, s4, g2, b2)


def kernel(atom_fea, nbr_fea, nbr_fea_idx, W_full, b_full,
           gamma1, beta1, gamma2, beta2):
    w_cat = jnp.concatenate([W_full[:A], W_full[A:2 * A]], axis=1)
    we = W_full[2 * A:]
    b2d = b_full[None, :]
    g1 = gamma1.reshape(2, A)
    b1 = beta1.reshape(2, A)
    g2 = gamma2[None, :]
    bt2 = beta2[None, :]

    a, pn = _prep(atom_fea, w_cat, b2d)
    idx_t = jnp.transpose(nbr_fea_idx).astype(jnp.int32)
    nft = jnp.transpose(nbr_fea, (1, 0, 2))

    g3s, s1s, s2s = [], [], []
    for h in range(NCHUNKS):
        idx_h = lax.slice(idx_t, (0, h * NHALF), (M, (h + 1) * NHALF))
        g3 = _gather(idx_h.reshape(-1), pn).reshape(M, NHALF, A)
        s1h, s2h = _make_stats(h)(g3, nft, a, we)
        g3s.append(g3)
        s1s.append(s1h)
        s2s.append(s2h)
    s1 = sum(s1s[1:], s1s[0])
    s2 = sum(s2s[1:], s2s[0])

    sms, s3s, s4s = [], [], []
    for h in range(NCHUNKS):
        smh, s3h, s4h = _make_pass2(h)(g3s[h], nft, a, we, s1, s2, g1, b1)
        sms.append(smh)
        s3s.append(s3h)
        s4s.append(s4h)
    sm = jnp.concatenate(sms, axis=0)
    s3 = sum(s3s[1:], s3s[0])
    s4 = sum(s4s[1:], s4s[0])
    return _final(sm, atom_fea, s3, s4, g2, bt2)


# speedup vs baseline: 1.0102x; 1.0102x over previous
import functools

import jax
import jax.numpy as jnp
import numpy as np
from jax import lax
from jax.experimental import pallas as pl
from jax.experimental.pallas import tpu as pltpu
from jax.experimental.pallas import tpu_sc as plsc

A = 128
E = 16
M = 32
N = 10000
NE = N * M
D = 2 * A
EPS = 1e-5
LOG2E = 1.4426950408889634

NCHUNKS = 2
NHALF = N // NCHUNKS
NEH = NHALF * M

NW = 32


def _softplus(x):
    return jnp.maximum(x, 0.0) + jnp.log(1.0 + jnp.exp(-jnp.abs(x)))


def _pack_halves(zf, zc):
    uf = jax.lax.bitcast_convert_type(zf, jnp.uint32)
    uc = jax.lax.bitcast_convert_type(zc, jnp.uint32)
    hi = (uf + np.uint32(0x8000)) & np.uint32(0xFFFF0000)
    lo = (uc + np.uint32(0x8000)) >> 16
    return hi | lo


def _unpack_halves(w):
    gf = jax.lax.bitcast_convert_type(w & np.uint32(0xFFFF0000), jnp.float32)
    gc = jax.lax.bitcast_convert_type(w << 16, jnp.float32)
    return gf, gc


def _prep_body(af_ref, w_ref, b_ref, a_ref, pn_ref):
    p = jnp.dot(af_ref[...], w_ref[...], preferred_element_type=jnp.float32)
    a_ref[...] = p[:, :D] + b_ref[...]
    pn = p[:, D:]
    pn_ref[...] = _pack_halves(pn[:, :A], pn[:, A:])


def _prep(atom_fea, w_cat, b2d):
    bn = 2000
    return pl.pallas_call(
        _prep_body,
        grid=(N // bn,),
        in_specs=[
            pl.BlockSpec((bn, A), lambda i: (i, 0)),
            pl.BlockSpec((A, 2 * D), lambda i: (0, 0)),
            pl.BlockSpec((1, D), lambda i: (0, 0)),
        ],
        out_specs=[
            pl.BlockSpec((bn, D), lambda i: (i, 0)),
            pl.BlockSpec((bn, A), lambda i: (i, 0)),
        ],
        out_shape=[
            jax.ShapeDtypeStruct((N, D), jnp.float32),
            jax.ShapeDtypeStruct((N, A), jnp.uint32),
        ],
    )(atom_fea, w_cat, b2d)


EPW = NEH // NW
KCH = 200
NCH = EPW // KCH


def _gather_body(idx_hbm, pn_hbm, out_hbm, idx_v, rows0, rows1, sem0, sem1):
    wid = lax.axis_index("s") * 2 + lax.axis_index("c")
    base = wid * EPW
    pltpu.sync_copy(idx_hbm.at[pl.ds(base, EPW)], idx_v)

    pltpu.async_copy(pn_hbm.at[idx_v.at[pl.ds(0, KCH)]], rows0, sem0)

    def pair(i, carry):
        c0 = 2 * i
        c1 = c0 + 1
        pltpu.async_copy(pn_hbm.at[idx_v.at[pl.ds(c1 * KCH, KCH)]], rows1, sem1)
        pltpu.make_async_copy(
            pn_hbm.at[idx_v.at[pl.ds(c0 * KCH, KCH)]], rows0, sem0
        ).wait()
        pltpu.sync_copy(rows0, out_hbm.at[pl.ds(base + c0 * KCH, KCH)])
        pltpu.async_copy(
            pn_hbm.at[idx_v.at[pl.ds((c0 + 2) * KCH, KCH)]], rows0, sem0
        )
        pltpu.make_async_copy(
            pn_hbm.at[idx_v.at[pl.ds(c1 * KCH, KCH)]], rows1, sem1
        ).wait()
        pltpu.sync_copy(rows1, out_hbm.at[pl.ds(base + c1 * KCH, KCH)])
        return carry

    lax.fori_loop(0, (NCH - 1) // 2, pair, 0)
    cL = NCH - 1
    pltpu.make_async_copy(
        pn_hbm.at[idx_v.at[pl.ds(cL * KCH, KCH)]], rows0, sem0
    ).wait()
    pltpu.sync_copy(rows0, out_hbm.at[pl.ds(base + cL * KCH, KCH)])


def _gather(idx_flat, pn):
    mesh = plsc.VectorSubcoreMesh(core_axis_name="c", subcore_axis_name="s")
    k = functools.partial(
        pl.kernel,
        out_type=jax.ShapeDtypeStruct((NEH, A), jnp.uint32),
        mesh=mesh,
        scratch_types=[
            pltpu.VMEM((EPW,), jnp.int32),
            pltpu.VMEM((KCH, A), jnp.uint32),
            pltpu.VMEM((KCH, A), jnp.uint32),
            pltpu.SemaphoreType.DMA,
            pltpu.SemaphoreType.DMA,
        ],
    )(_gather_body)
    return k(idx_flat, pn)


BN3 = 200
GRID3 = NHALF // BN3
BE3 = BN3 * M


def _make_stats(h):
    nb = h * (NHALF // BN3)

    def body(g_ref, nf_ref, a_ref, we_ref, s1_ref, s2_ref, acc1, acc2):
        gf, gc = _unpack_halves(g_ref[...])
        nf2 = jnp.reshape(nf_ref[...], (BE3, E))
        e = jnp.dot(nf2, we_ref[...], preferred_element_type=jnp.float32)
        a = a_ref[...]
        zf = jnp.reshape(
            gf + jnp.reshape(e[:, :A], (M, BN3, A)) + a[None, :, :A], (BE3, A))
        zc = jnp.reshape(
            gc + jnp.reshape(e[:, A:], (M, BN3, A)) + a[None, :, A:], (BE3, A))

        @pl.when(pl.program_id(0) == 0)
        def _():
            acc1[...] = jnp.zeros_like(acc1)
            acc2[...] = jnp.zeros_like(acc2)

        acc1[0:1, :] += jnp.sum(zf, axis=0, keepdims=True)
        acc1[1:2, :] += jnp.sum(zc, axis=0, keepdims=True)
        acc2[0:1, :] += jnp.sum(zf * zf, axis=0, keepdims=True)
        acc2[1:2, :] += jnp.sum(zc * zc, axis=0, keepdims=True)

        @pl.when(pl.program_id(0) == GRID3 - 1)
        def _():
            s1_ref[...] = acc1[...]
            s2_ref[...] = acc2[...]

    def call(g3, nft, a, we):
        return pl.pallas_call(
            body,
            grid=(GRID3,),
            in_specs=[
                pl.BlockSpec((M, BN3, A), lambda i: (0, i, 0)),
                pl.BlockSpec((M, BN3, E), lambda i: (0, i + nb, 0)),
                pl.BlockSpec((BN3, D), lambda i: (i + h * (NHALF // BN3), 0)),
                pl.BlockSpec((E, D), lambda i: (0, 0)),
            ],
            out_specs=[
                pl.BlockSpec((2, A), lambda i: (0, 0)),
                pl.BlockSpec((2, A), lambda i: (0, 0)),
            ],
            out_shape=[
                jax.ShapeDtypeStruct((2, A), jnp.float32),
                jax.ShapeDtypeStruct((2, A), jnp.float32),
            ],
            scratch_shapes=[
                pltpu.VMEM((2, A), jnp.float32),
                pltpu.VMEM((2, A), jnp.float32),
            ],
        )(g3, nft, a, we)

    return call


BN = 200
GRID4 = NHALF // BN
BE = BN * M


def _make_pass2(h):
    nb = h * (NHALF // BN)

    def body(g_ref, nf_ref, a_ref, we_ref, s1_ref, s2_ref, g1_ref, b1_ref,
             sm_ref, s3_ref, s4_ref, acc3, acc4):
        mu = s1_ref[...] * (1.0 / NE)
        var = s2_ref[...] * (1.0 / NE) - mu * mu
        s = g1_ref[...] * lax.rsqrt(var + EPS)
        t = b1_ref[...] - mu * s
        sx = jnp.concatenate([s[0:1, :] * (-LOG2E), s[1:2, :] * LOG2E], axis=1)
        tx = jnp.concatenate([t[0:1, :] * (-LOG2E), t[1:2, :] * LOG2E], axis=1)
        we_s = we_ref[...] * sx
        a2 = a_ref[...] * sx + tx

        gf, gc = _unpack_halves(g_ref[...])
        nf2 = jnp.reshape(nf_ref[...], (BE, E))
        e = jnp.dot(nf2, we_s, preferred_element_type=jnp.float32)
        yf3 = (gf * sx[:1, None, :A] + jnp.reshape(e[:, :A], (M, BN, A))
               + a2[None, :, :A])
        yc3 = (gc * sx[:1, None, A:] + jnp.reshape(e[:, A:], (M, BN, A))
               + a2[None, :, A:])
        prod = jnp.log2(1.0 + jnp.exp2(yc3)) / (1.0 + jnp.exp2(yf3))
        sm = jnp.sum(prod, axis=0)
        sm_ref[...] = sm

        @pl.when(pl.program_id(0) == 0)
        def _():
            acc3[...] = jnp.zeros_like(acc3)
            acc4[...] = jnp.zeros_like(acc4)

        acc3[...] += jnp.sum(sm, axis=0, keepdims=True)
        acc4[...] += jnp.sum(sm * sm, axis=0, keepdims=True)

        @pl.when(pl.program_id(0) == GRID4 - 1)
        def _():
            s3_ref[...] = acc3[...]
            s4_ref[...] = acc4[...]

    def call(g3, nft, a, we, s1, s2, g1, b1):
        return pl.pallas_call(
            body,
            grid=(GRID4,),
            in_specs=[
                pl.BlockSpec((M, BN, A), lambda i: (0, i, 0)),
                pl.BlockSpec((M, BN, E), lambda i: (0, i + nb, 0)),
                pl.BlockSpec((BN, D), lambda i: (i + h * (NHALF // BN), 0)),
                pl.BlockSpec((E, D), lambda i: (0, 0)),
                pl.BlockSpec((2, A), lambda i: (0, 0)),
                pl.BlockSpec((2, A), lambda i: (0, 0)),
                pl.BlockSpec((2, A), lambda i: (0, 0)),
                pl.BlockSpec((2, A), lambda i: (0, 0)),
            ],
            out_specs=[
                pl.BlockSpec((BN, A), lambda i: (i, 0)),
                pl.BlockSpec((1, A), lambda i: (0, 0)),
                pl.BlockSpec((1, A), lambda i: (0, 0)),
            ],
            out_shape=[
                jax.ShapeDtypeStruct((NHALF, A), jnp.float32),
                jax.ShapeDtypeStruct((1, A), jnp.float32),
                jax.ShapeDtypeStruct((1, A), jnp.float32),
            ],
            scratch_shapes=[
                pltpu.VMEM((1, A), jnp.float32),
                pltpu.VMEM((1, A), jnp.float32),
            ],
        )(g3, nft, a, we, s1, s2, g1, b1)

    return call


def _final_body(sm_ref, af_ref, s3_ref, s4_ref, g2_ref, b2_ref, out_ref):
    mu = s3_ref[...] * (1.0 / N)
    var = s4_ref[...] * (1.0 / N) - mu * mu
    s = g2_ref[...] * lax.rsqrt(var + EPS)
    t = b2_ref[...] - mu * s
    out_ref[...] = _softplus(af_ref[...] + sm_ref[...] * s + t)


def _final(sm, atom_fea, s3, s4, g2, b2):
    bn5 = 2000
    return pl.pallas_call(
        _final_body,
        grid=(N // bn5,),
        in_specs=[
            pl.BlockSpec((bn5, A), lambda i: (i, 0)),
            pl.BlockSpec((bn5, A), lambda i: (i, 0)),
            pl.BlockSpec((1, A), lambda i: (0, 0)),
            pl.BlockSpec((1, A), lambda i: (0, 0)),
            pl.BlockSpec((1, A), lambda i: (0, 0)),
            pl.BlockSpec((1, A), lambda i: (0, 0)),
        ],
        out_specs=pl.BlockSpec((bn5, A), lambda i: (i, 0)),
        out_shape=jax.ShapeDtypeStruct((N, A), jnp.float32),
    )(sm, atom_fea, s3, s4, g2, b2)


def kernel(atom_fea, nbr_fea, nbr_fea_idx, W_full, b_full,
           gamma1, beta1, gamma2, beta2):
    w_cat = jnp.concatenate([W_full[:A], W_full[A:2 * A]], axis=1)
    we = W_full[2 * A:]
    b2d = b_full[None, :]
    g1 = gamma1.reshape(2, A)
    b1 = beta1.reshape(2, A)
    g2 = gamma2[None, :]
    bt2 = beta2[None, :]

    a, pn = _prep(atom_fea, w_cat, b2d)
    idx_t = jnp.transpose(nbr_fea_idx).astype(jnp.int32)
    nft = jnp.transpose(nbr_fea, (1, 0, 2))

    g3s, s1s, s2s = [], [], []
    for h in range(NCHUNKS):
        idx_h = lax.slice(idx_t, (0, h * NHALF), (M, (h + 1) * NHALF))
        g3 = _gather(idx_h.reshape(-1), pn).reshape(M, NHALF, A)
        s1h, s2h = _make_stats(h)(g3, nft, a, we)
        g3s.append(g3)
        s1s.append(s1h)
        s2s.append(s2h)
    s1 = sum(s1s[1:], s1s[0])
    s2 = sum(s2s[1:], s2s[0])

    sms, s3s, s4s = [], [], []
    for h in range(NCHUNKS):
        smh, s3h, s4h = _make_pass2(h)(g3s[h], nft, a, we, s1, s2, g1, b1)
        sms.append(smh)
        s3s.append(s3h)
        s4s.append(s4h)
    sm = jnp.concatenate(sms, axis=0)
    s3 = sum(s3s[1:], s3s[0])
    s4 = sum(s4s[1:], s4s[0])
    return _final(sm, atom_fea, s3, s4, g2, bt2)
